# baseline (speedup 1.0000x reference)
import functools
import math

import jax
import jax.numpy as jnp
from jax import lax
from jax.experimental import pallas as pl
from jax.experimental.pallas import tpu as pltpu
from jax.experimental.pallas import tpu_sc as plsc

_NUM_LEVELS = 16
_MIN_RES = 16
_MAX_RES = 2048
_GROWTH = math.exp((math.log(_MAX_RES) - math.log(_MIN_RES)) / (_NUM_LEVELS - 1))
_RES = [int(_MIN_RES * _GROWTH ** i) for i in range(_NUM_LEVELS)]
_MOD = 4096
_Q1 = 2654435761 % _MOD
_Q2 = 805459861 % _MOD

_NC = 2
_NS = 16
_NW = _NC * _NS
_LANES = 16

_N = 524288
_PTS_PER_W = _N // _NW
_BLK = 1024
_NBLK = _PTS_PER_W // _BLK
_NVEC = _BLK // _LANES
_LPP = 8
_PT_TILES = _N // 128
_BLK_TILES = _BLK // 128


def _make_kernel():
    mesh = plsc.VectorSubcoreMesh(
        core_axis_name="c", subcore_axis_name="s",
        num_cores=_NC, num_subcores=_NS)

    @functools.partial(
        pl.kernel,
        mesh=mesh,
        out_type=jax.ShapeDtypeStruct((4, _PT_TILES, 8, 128), jnp.float32),
        compiler_params=pltpu.CompilerParams(use_tc_tiling_on_sc=False,
                                             needs_layout_passes=False),
        scratch_types=[
            pltpu.VMEM((_LPP * 2 * _MOD,), jnp.float32),
            pltpu.VMEM((2, _BLK * 3), jnp.float32),
            pltpu.VMEM((2, 2, _BLK_TILES, 8, 128), jnp.float32),
            pltpu.SemaphoreType.DMA,
            pltpu.SemaphoreType.DMA,
            pltpu.SemaphoreType.DMA,
            pltpu.SemaphoreType.DMA,
        ],
    )
    def hash_encode(pos_hbm, hot_hbm, out_hbm, tab_v, pos_v, out_v,
                    sem_in0, sem_in1, sem_out0, sem_out1):
        wid = lax.axis_index("s") * _NC + lax.axis_index("c")
        base = wid * _PTS_PER_W
        sems_in = (sem_in0, sem_in1)
        sems_out = (sem_out0, sem_out1)

        def start_in(blk, b):
            row0 = base + blk * _BLK
            pltpu.async_copy(pos_hbm.at[pl.ds(row0 * 3, _BLK * 3)],
                             pos_v.at[jnp.int32(b)], sems_in[b])

        def wait_in(b):
            pltpu.make_async_copy(pos_hbm.at[pl.ds(base, _BLK * 3)],
                                  pos_v.at[jnp.int32(b)], sems_in[b]).wait()

        def start_out(blk, b, p):
            ptile0 = lax.div(base + blk * _BLK, jnp.int32(128))
            for k in range(2):
                pltpu.async_copy(
                    out_v.at[jnp.int32(b), jnp.int32(k)],
                    out_hbm.at[jnp.int32(2 * p + k), pl.ds(ptile0, _BLK_TILES)],
                    sems_out[b])

        def wait_out(b, p):
            for k in range(2):
                pltpu.make_async_copy(
                    out_v.at[jnp.int32(b), jnp.int32(k)],
                    out_hbm.at[jnp.int32(2 * p + k), pl.ds(jnp.int32(0), _BLK_TILES)],
                    sems_out[b]).wait()

        half = _LPP * 2 * _MOD
        for p in range(2):
            pltpu.sync_copy(hot_hbm.at[pl.ds(half * p, half)], tab_v)
            start_in(jnp.int32(0), 0)

            def compute_block(blk, b, p=p):
                def tile_body(t, _, p=p, b=b):
                    i3 = lax.iota(jnp.int32, _LANES) * jnp.int32(3)
                    toff = t * jnp.int32(128)
                    for u in range(8):
                        off = toff + jnp.int32(u * _LANES)
                        ix = i3 + off * jnp.int32(3)
                        pvb = pos_v.at[jnp.int32(b)]
                        x = plsc.load_gather(pvb, [ix])
                        y = plsc.load_gather(pvb, [ix + jnp.int32(1)])
                        z = plsc.load_gather(pvb, [ix + jnp.int32(2)])
                        for j in range(_LPP):
                            r = _RES[_LPP * p + j]
                            cx = (x * jnp.float32(r - 1)).astype(jnp.int32)
                            cy = (y * jnp.float32(r - 1)).astype(jnp.int32)
                            cz = (z * jnp.float32(r - 1)).astype(jnp.int32)
                            h = (cx + cy * _Q1 + cz * _Q2) & (_MOD - 1)
                            for fe in range(2):
                                g = plsc.load_gather(
                                    tab_v.at[pl.ds((2 * j + fe) * _MOD, _MOD)],
                                    [h])
                                f = 2 * j + fe
                                out_v[jnp.int32(b), jnp.int32(f // 8), t,
                                      jnp.int32(f % 8),
                                      pl.ds(u * _LANES, _LANES)] = g
                    return jnp.int32(0)

                lax.fori_loop(jnp.int32(0), jnp.int32(_BLK_TILES), tile_body,
                              jnp.int32(0))

            def pair_body(pair, _, p=p):
                for b in range(2):
                    blk = pair * jnp.int32(2) + jnp.int32(b)
                    wait_in(b)

                    @pl.when(blk + 1 < _NBLK)
                    def _():
                        start_in(blk + 1, 1 - b)

                    @pl.when(pair >= 1)
                    def _():
                        wait_out(b, p)

                    compute_block(blk, b)
                    start_out(blk, b, p)
                return jnp.int32(0)

            lax.fori_loop(jnp.int32(0), jnp.int32(_NBLK // 2), pair_body,
                          jnp.int32(0))
            for b in range(2):
                wait_out(b, p)

    return hash_encode


_KERNEL_CACHE = []


def kernel(positions, tables):
    if not _KERNEL_CACHE:
        _KERNEL_CACHE.append(_make_kernel())
    pos_flat = positions.reshape(-1)
    hot = jnp.stack([t[:_MOD] for t in tables]).transpose(0, 2, 1).reshape(-1)
    out4d = _KERNEL_CACHE[0](pos_flat, hot)
    return out4d.transpose(1, 3, 0, 2).reshape(_N, 2 * _NUM_LEVELS)


# speedup vs baseline: 3.8168x; 3.8168x over previous
import functools
import math

import jax
import jax.numpy as jnp
from jax import lax
from jax.experimental import pallas as pl
from jax.experimental.pallas import tpu as pltpu
from jax.experimental.pallas import tpu_sc as plsc

_NUM_LEVELS = 16
_MIN_RES = 16
_MAX_RES = 2048
_GROWTH = math.exp((math.log(_MAX_RES) - math.log(_MIN_RES)) / (_NUM_LEVELS - 1))
_RES = [int(_MIN_RES * _GROWTH ** i) for i in range(_NUM_LEVELS)]
_MOD = 4096
_Q1 = 2654435761 % _MOD
_Q2 = 805459861 % _MOD

_NC = 2
_NS = 16
_NW = _NC * _NS
_LANES = 16

_N = 524288
_PTS_PER_W = _N // _NW
_BLK = 1024
_NBLK = _PTS_PER_W // _BLK
_NVEC = _BLK // _LANES
_LPP = 8
_PT_TILES = _N // 128
_BLK_TILES = _BLK // 128


def _make_kernel():
    mesh = plsc.VectorSubcoreMesh(
        core_axis_name="c", subcore_axis_name="s",
        num_cores=_NC, num_subcores=_NS)

    @functools.partial(
        pl.kernel,
        mesh=mesh,
        out_type=jax.ShapeDtypeStruct((4, _PT_TILES, 8, 128), jnp.float32),
        compiler_params=pltpu.CompilerParams(use_tc_tiling_on_sc=False,
                                             needs_layout_passes=False),
        scratch_types=[
            pltpu.VMEM((_LPP * 2 * _MOD,), jnp.float32),
            pltpu.VMEM((2, 3, _BLK), jnp.float32),
            pltpu.VMEM((2, 2, _BLK_TILES, 8, 128), jnp.float32),
            pltpu.SemaphoreType.DMA,
            pltpu.SemaphoreType.DMA,
            pltpu.SemaphoreType.DMA,
            pltpu.SemaphoreType.DMA,
        ],
    )
    def hash_encode(pos_hbm, hot_hbm, out_hbm, tab_v, pos_v, out_v,
                    sem_in0, sem_in1, sem_out0, sem_out1):
        wid = lax.axis_index("s") * _NC + lax.axis_index("c")
        base = wid * _PTS_PER_W
        sems_in = (sem_in0, sem_in1)
        sems_out = (sem_out0, sem_out1)

        def start_in(blk, b):
            row0 = base + blk * _BLK
            pltpu.async_copy(pos_hbm.at[:, pl.ds(row0, _BLK)],
                             pos_v.at[jnp.int32(b)], sems_in[b])

        def wait_in(b):
            pltpu.make_async_copy(pos_hbm.at[:, pl.ds(base, _BLK)],
                                  pos_v.at[jnp.int32(b)], sems_in[b]).wait()

        def start_out(blk, b, p):
            ptile0 = lax.div(base + blk * _BLK, jnp.int32(128))
            for k in range(2):
                pltpu.async_copy(
                    out_v.at[jnp.int32(b), jnp.int32(k)],
                    out_hbm.at[jnp.int32(2 * p + k), pl.ds(ptile0, _BLK_TILES)],
                    sems_out[b])

        def wait_out(b, p):
            for k in range(2):
                pltpu.make_async_copy(
                    out_v.at[jnp.int32(b), jnp.int32(k)],
                    out_hbm.at[jnp.int32(2 * p + k), pl.ds(jnp.int32(0), _BLK_TILES)],
                    sems_out[b]).wait()

        half = _LPP * 2 * _MOD
        for p in range(2):
            pltpu.sync_copy(hot_hbm.at[pl.ds(half * p, half)], tab_v)
            start_in(jnp.int32(0), 0)

            def compute_block(blk, b, p=p):
                def tile_body(t, _, p=p, b=b):
                    toff = t * jnp.int32(128)
                    for u in range(8):
                        off = toff + jnp.int32(u * _LANES)
                        x = pos_v[jnp.int32(b), jnp.int32(0), pl.ds(off, _LANES)]
                        y = pos_v[jnp.int32(b), jnp.int32(1), pl.ds(off, _LANES)]
                        z = pos_v[jnp.int32(b), jnp.int32(2), pl.ds(off, _LANES)]
                        for j in range(_LPP):
                            r = _RES[_LPP * p + j]
                            cx = (x * jnp.float32(r - 1)).astype(jnp.int32)
                            cy = (y * jnp.float32(r - 1)).astype(jnp.int32)
                            cz = (z * jnp.float32(r - 1)).astype(jnp.int32)
                            h = (cx + cy * _Q1 + cz * _Q2) & (_MOD - 1)
                            for fe in range(2):
                                g = plsc.load_gather(
                                    tab_v.at[pl.ds((2 * j + fe) * _MOD, _MOD)],
                                    [h])
                                f = 2 * j + fe
                                out_v[jnp.int32(b), jnp.int32(f // 8), t,
                                      jnp.int32(f % 8),
                                      pl.ds(u * _LANES, _LANES)] = g
                    return jnp.int32(0)

                lax.fori_loop(jnp.int32(0), jnp.int32(_BLK_TILES), tile_body,
                              jnp.int32(0))

            def pair_body(pair, _, p=p):
                for b in range(2):
                    blk = pair * jnp.int32(2) + jnp.int32(b)
                    wait_in(b)

                    @pl.when(blk + 1 < _NBLK)
                    def _():
                        start_in(blk + 1, 1 - b)

                    @pl.when(pair >= 1)
                    def _():
                        wait_out(b, p)

                    compute_block(blk, b)
                    start_out(blk, b, p)
                return jnp.int32(0)

            lax.fori_loop(jnp.int32(0), jnp.int32(_NBLK // 2), pair_body,
                          jnp.int32(0))
            for b in range(2):
                wait_out(b, p)

    return hash_encode


_KERNEL_CACHE = []


def kernel(positions, tables):
    if not _KERNEL_CACHE:
        _KERNEL_CACHE.append(_make_kernel())
    pos_t = positions.T
    hot = jnp.stack([t[:_MOD] for t in tables]).transpose(0, 2, 1).reshape(-1)
    out4d = _KERNEL_CACHE[0](pos_t, hot)
    return out4d.transpose(1, 3, 0, 2).reshape(_N, 2 * _NUM_LEVELS)
